# baseline (speedup 1.0000x reference)
import jax
import jax.numpy as jnp
from jax.experimental import pallas as pl
from jax.experimental.pallas import tpu as pltpu

_HALF = 4096
_RSPLIT = 4
_RSUB = _HALF // _RSPLIT


def _make_body(batch):
    def body(pe_hbm, out_hbm, buf0, buf1,
             ra0, ra1, ra2, ra3, rb0, rb1, rb2, rb3, wsem0, wsem1):
        bufs = (buf0, buf1)
        rsems = ((ra0, ra1, ra2, ra3), (rb0, rb1, rb2, rb3))
        wsems = (wsem0, wsem1)

        def read_subs(half, parity):
            return [
                pltpu.make_async_copy(
                    pe_hbm.at[pl.ds(half * _HALF + k * _RSUB, _RSUB)],
                    bufs[parity].at[pl.ds(k * _RSUB, _RSUB)],
                    rsems[parity][k])
                for k in range(_RSPLIT)
            ]

        def write_copies(half, parity):
            return [
                pltpu.make_async_copy(
                    bufs[parity],
                    out_hbm.at[b, pl.ds(half * _HALF, _HALF)],
                    wsems[parity])
                for b in range(batch)
            ]

        r0 = read_subs(0, 0)
        for c in r0:
            c.start()
        for c in r0:
            c.wait()
        w0 = write_copies(0, 0)
        for c in w0:
            c.start()
        r1 = read_subs(1, 1)
        for c in r1:
            c.start()
        for c in r1:
            c.wait()
        w1 = write_copies(1, 1)
        for c in w1:
            c.start()
        for c in w0:
            c.wait()
        for c in w1:
            c.wait()
    return body


def kernel(inputs, position_embeddings):
    batch, seq, width = inputs.shape
    pe = position_embeddings[:seq, :]
    out = pl.pallas_call(
        _make_body(batch),
        grid=(1,),
        in_specs=[pl.BlockSpec(memory_space=pl.ANY)],
        out_specs=pl.BlockSpec(memory_space=pl.ANY),
        out_shape=jax.ShapeDtypeStruct((batch, seq, width), jnp.float32),
        scratch_shapes=[
            pltpu.VMEM((_HALF, width), jnp.float32),
            pltpu.VMEM((_HALF, width), jnp.float32),
        ] + [pltpu.SemaphoreType.DMA] * 10,
    )(pe)
    return out
